# baseline (speedup 1.0000x reference)
import jax
import jax.numpy as jnp
from jax import lax
from jax.experimental import pallas as pl
from jax.experimental.pallas import tpu as pltpu
from jax.experimental.pallas import tpu_sc as plsc

N_HID = 256
E = 160000
TCR = 79872
BLK = 1024
SCR = E - TCR
NC = 2
NS = 16
NW = NC * NS
BPW = SCR // NW
CH = 104
NCH = -(-BPW // CH)
TS = BPW - (NCH - 1) * CH
NBUF = 4
PF = 2


def _fuse_body(emb_ref, w_ref, b_ref, out_ref):
    out_ref[...] = lax.dot_general(
        emb_ref[...], w_ref[...],
        (((1,), (1,)), ((), ())),
        preferred_element_type=jnp.float32,
        precision=lax.Precision.HIGHEST,
    ) + b_ref[...]


def _fuse_table(emb_table, W, b):
    m, n = emb_table.shape
    return pl.pallas_call(
        _fuse_body,
        out_shape=jax.ShapeDtypeStruct((m, n), jnp.float32),
    )(emb_table, W, b.reshape(1, n))


def _gather_body(table_hbm, idx_hbm, out_hbm, idx_v, rows_v, gs, ws):
    wid = lax.axis_index("s") * NC + lax.axis_index("c")
    base = pl.multiple_of(TCR + wid * BPW, 8)
    pltpu.sync_copy(idx_hbm.at[wid], idx_v)

    def gather(c):
        b = c % NBUF
        return pltpu.make_async_copy(
            table_hbm.at[idx_v.at[c]], rows_v.at[b], gs[b])

    def write(c):
        b = c % NBUF
        n = TS if c == NCH - 1 else CH
        return pltpu.make_async_copy(
            rows_v.at[b, pl.ds(0, n)],
            out_hbm.at[pl.ds(pl.multiple_of(base + c * CH, 8), n)], ws[b])

    for c in range(min(PF, NCH)):
        gather(c).start()
    for c in range(NCH):
        pc = c + PF
        if pc < NCH:
            if pc - NBUF >= 0:
                write(pc - NBUF).wait()
            gather(pc).start()
        gather(c).wait()
        write(c).start()
    for c in range(max(0, NCH - NBUF), NCH):
        write(c).wait()


def _sc_gather(table, idx):
    mesh = plsc.VectorSubcoreMesh(
        core_axis_name="c", subcore_axis_name="s",
        num_cores=NC, num_subcores=NS)
    return pl.kernel(
        _gather_body,
        out_type=jax.ShapeDtypeStruct((E, N_HID), jnp.float32),
        mesh=mesh,
        scratch_types=[
            pltpu.VMEM((NCH, CH), jnp.int32),
            pltpu.VMEM((NBUF, CH, N_HID), jnp.float32),
            [pltpu.SemaphoreType.DMA] * NBUF,
            [pltpu.SemaphoreType.DMA] * NBUF,
        ],
    )(table, idx)


def _tc_gather_body(idx_ref, table_ref, partial_ref, out_ref):
    del partial_ref
    onehot = (idx_ref[...] ==
              lax.broadcasted_iota(jnp.int32, (BLK, 256), 1)
              ).astype(jnp.float32)
    out_ref[...] = lax.dot_general(
        onehot, table_ref[...],
        (((1,), (0,)), ((), ())),
        preferred_element_type=jnp.float32,
        precision=lax.Precision.HIGHEST,
    )


def _tc_gather(table256, idx, partial):
    return pl.pallas_call(
        _tc_gather_body,
        grid=(TCR // BLK,),
        in_specs=[
            pl.BlockSpec((BLK, 1), lambda i: (i, 0)),
            pl.BlockSpec((256, N_HID), lambda i: (0, 0)),
            pl.BlockSpec(memory_space=pl.ANY),
        ],
        out_specs=pl.BlockSpec((BLK, N_HID), lambda i: (i, 0)),
        out_shape=jax.ShapeDtypeStruct((E, N_HID), jnp.float32),
        input_output_aliases={2: 0},
    )(idx.reshape(TCR, 1), table256, partial)


def kernel(t, emb_table, W, b):
    fused = _fuse_table(emb_table, W, b)
    sc_idx = jnp.pad(t[TCR:].reshape(NW, BPW),
                     ((0, 0), (0, NCH * CH - BPW)))
    partial = _sc_gather(fused, sc_idx.reshape(NW, NCH, CH))
    table256 = jnp.pad(fused, ((0, 256 - fused.shape[0]), (0, 0)))
    return _tc_gather(table256, t[:TCR], partial)


# speedup vs baseline: 1.3740x; 1.3740x over previous
import jax
import jax.numpy as jnp
from jax import lax
from jax.experimental import pallas as pl
from jax.experimental.pallas import tpu as pltpu
from jax.experimental.pallas import tpu_sc as plsc

N_HID = 256
E = 160000
TCR = 73728
BLK = 1024
SCR = E - TCR
NC = 2
NS = 16
NW = NC * NS
BPW = SCR // NW
CH = 104
NCH = -(-BPW // CH)
TS = BPW - (NCH - 1) * CH
NBUF = 4
PF = 2


def _fuse_body(emb_ref, w_ref, b_ref, out_ref):
    out_ref[...] = lax.dot_general(
        emb_ref[...], w_ref[...],
        (((1,), (1,)), ((), ())),
        preferred_element_type=jnp.float32,
        precision=lax.Precision.HIGHEST,
    ) + b_ref[...]


def _fuse_table(emb_table, W, b):
    m, n = emb_table.shape
    return pl.pallas_call(
        _fuse_body,
        out_shape=jax.ShapeDtypeStruct((m, n), jnp.float32),
    )(emb_table, W, b.reshape(1, n))


def _gather_body(table_hbm, idx_hbm, out_hbm, idx_v, rows_v, gs, ws):
    wid = lax.axis_index("s") * NC + lax.axis_index("c")
    base = pl.multiple_of(TCR + wid * BPW, 8)
    pltpu.sync_copy(idx_hbm.at[wid], idx_v)

    def gather(c):
        b = c % NBUF
        return pltpu.make_async_copy(
            table_hbm.at[idx_v.at[c]], rows_v.at[b], gs[b])

    def write(c):
        b = c % NBUF
        n = TS if c == NCH - 1 else CH
        return pltpu.make_async_copy(
            rows_v.at[b, pl.ds(0, n)],
            out_hbm.at[pl.ds(pl.multiple_of(base + c * CH, 8), n)], ws[b])

    for c in range(min(PF, NCH)):
        gather(c).start()
    for c in range(NCH):
        pc = c + PF
        if pc < NCH:
            if pc - NBUF >= 0:
                write(pc - NBUF).wait()
            gather(pc).start()
        gather(c).wait()
        write(c).start()
    for c in range(max(0, NCH - NBUF), NCH):
        write(c).wait()


def _sc_gather(table, idx):
    mesh = plsc.VectorSubcoreMesh(
        core_axis_name="c", subcore_axis_name="s",
        num_cores=NC, num_subcores=NS)
    return pl.kernel(
        _gather_body,
        out_type=jax.ShapeDtypeStruct((E, N_HID), jnp.float32),
        mesh=mesh,
        scratch_types=[
            pltpu.VMEM((NCH, CH), jnp.int32),
            pltpu.VMEM((NBUF, CH, N_HID), jnp.float32),
            [pltpu.SemaphoreType.DMA] * NBUF,
            [pltpu.SemaphoreType.DMA] * NBUF,
        ],
    )(table, idx)


def _tc_gather_body(idx_ref, table_ref, partial_ref, out_ref):
    del partial_ref
    onehot = (idx_ref[...] ==
              lax.broadcasted_iota(jnp.int32, (BLK, 256), 1)
              ).astype(jnp.float32)
    out_ref[...] = lax.dot_general(
        onehot, table_ref[...],
        (((1,), (0,)), ((), ())),
        preferred_element_type=jnp.float32,
        precision=lax.Precision.HIGHEST,
    )


def _tc_gather(table256, idx, partial):
    return pl.pallas_call(
        _tc_gather_body,
        grid=(TCR // BLK,),
        in_specs=[
            pl.BlockSpec((BLK, 1), lambda i: (i, 0)),
            pl.BlockSpec((256, N_HID), lambda i: (0, 0)),
            pl.BlockSpec(memory_space=pl.ANY),
        ],
        out_specs=pl.BlockSpec((BLK, N_HID), lambda i: (i, 0)),
        out_shape=jax.ShapeDtypeStruct((E, N_HID), jnp.float32),
        input_output_aliases={2: 0},
    )(idx.reshape(TCR, 1), table256, partial)


def kernel(t, emb_table, W, b):
    fused = _fuse_table(emb_table, W, b)
    sc_idx = jnp.pad(t[TCR:].reshape(NW, BPW),
                     ((0, 0), (0, NCH * CH - BPW)))
    partial = _sc_gather(fused, sc_idx.reshape(NW, NCH, CH))
    table256 = jnp.pad(fused, ((0, 256 - fused.shape[0]), (0, 0)))
    return _tc_gather(table256, t[:TCR], partial)
